# Initial kernel scaffold; baseline (speedup 1.0000x reference)
import jax
import jax.numpy as jnp
from jax import lax
from jax.experimental import pallas as pl
from jax.experimental.pallas import tpu as pltpu
from jax.experimental.pallas import tpu_sc as plsc

PERIODS = 288
WEEKEND = 7
P_DIM = 24
W_DIM = 24
OUT_DIM = P_DIM + W_DIM
N_ROWS = 64 * 12 * 1000
NW = 32
PER_W = N_ROWS // NW
SUB = 96
NSUB = 5
B_BLK = SUB * NSUB
N_BLK = PER_W // B_BLK
GROUPS = B_BLK // 16


def _build_table_kernel(p_ref, w_ref, o_ref):
    pe = jnp.broadcast_to(p_ref[:][:, None, :], (PERIODS, WEEKEND, P_DIM))
    we = jnp.broadcast_to(w_ref[:][None, :, :], (PERIODS, WEEKEND, W_DIM))
    o_ref[:] = jnp.concatenate([pe, we], axis=-1)


def _build_fused_table(periods_embedding, weekend_embedding):
    fused3 = pl.pallas_call(
        _build_table_kernel,
        out_shape=jax.ShapeDtypeStruct((PERIODS, WEEKEND, OUT_DIM), jnp.float32),
    )(periods_embedding, weekend_embedding)
    return fused3.reshape(PERIODS * WEEKEND, OUT_DIM)


def _sc_body(x_hbm, tab_hbm, out_hbm, x_v, idx_v, rows_v, gsem, osem):
    wid = lax.axis_index("s") * 2 + lax.axis_index("c")
    iota = lax.iota(jnp.int32, 16)
    col1 = jnp.full((16,), 1, jnp.int32)
    col2 = jnp.full((16,), 2, jnp.int32)

    def block(j, carry):
        base = wid * PER_W + j * B_BLK
        pltpu.sync_copy(x_hbm.at[pl.ds(base, B_BLK)], x_v)
        for g in range(GROUPS):
            rows = g * 16 + iota
            pv = plsc.load_gather(x_v, [rows, col1])
            wv = plsc.load_gather(x_v, [rows, col2])
            pi = jnp.minimum((pv * float(PERIODS)).astype(jnp.int32), PERIODS - 1)
            wi = jnp.minimum(wv.astype(jnp.int32), WEEKEND - 1)
            idx_v[g // 6, pl.ds((g % 6) * 16, 16)] = pi * WEEKEND + wi
        descs = [
            pltpu.async_copy(
                tab_hbm.at[idx_v.at[r]], rows_v.at[pl.ds(r * SUB, SUB)], gsem
            )
            for r in range(NSUB)
        ]
        for d in descs:
            d.wait()
        pltpu.async_copy(rows_v, out_hbm.at[pl.ds(base, B_BLK)], osem).wait()
        return carry

    lax.fori_loop(0, N_BLK, block, 0)


@jax.jit
def _encode(x_flat, fused_table):
    mesh = plsc.VectorSubcoreMesh(core_axis_name="c", subcore_axis_name="s")
    return pl.kernel(
        _sc_body,
        out_type=jax.ShapeDtypeStruct((N_ROWS, OUT_DIM), jnp.float32),
        mesh=mesh,
        scratch_types=[
            pltpu.VMEM((B_BLK, 3), jnp.float32),
            pltpu.VMEM((NSUB, SUB), jnp.int32),
            pltpu.VMEM((B_BLK, OUT_DIM), jnp.float32),
            pltpu.SemaphoreType.DMA,
            pltpu.SemaphoreType.DMA,
        ],
    )(x_flat, fused_table)


def kernel(x, periods_embedding, weekend_embedding):
    b, t, n, _ = x.shape
    fused = _build_fused_table(periods_embedding, weekend_embedding)
    x_flat = x.reshape(b * t * n, 3)
    out = _encode(x_flat, fused)
    return out.reshape(b, t, n, OUT_DIM)


# speedup vs baseline: 1.5266x; 1.5266x over previous
import jax
import jax.numpy as jnp
from jax import lax
from jax.experimental import pallas as pl
from jax.experimental.pallas import tpu as pltpu
from jax.experimental.pallas import tpu_sc as plsc

PERIODS = 288
WEEKEND = 7
P_DIM = 24
W_DIM = 24
OUT_DIM = P_DIM + W_DIM
N_ROWS = 64 * 12 * 1000
NW = 32
PER_W = N_ROWS // NW
SUB = 96
NSUB = 5
B_BLK = SUB * NSUB
N_BLK = PER_W // B_BLK
GROUPS = B_BLK // 16


def _build_table_kernel(p_ref, w_ref, o_ref):
    pe = jnp.broadcast_to(p_ref[:][:, None, :], (PERIODS, WEEKEND, P_DIM))
    we = jnp.broadcast_to(w_ref[:][None, :, :], (PERIODS, WEEKEND, W_DIM))
    o_ref[:] = jnp.concatenate([pe, we], axis=-1)


def _build_fused_table(periods_embedding, weekend_embedding):
    fused3 = pl.pallas_call(
        _build_table_kernel,
        out_shape=jax.ShapeDtypeStruct((PERIODS, WEEKEND, OUT_DIM), jnp.float32),
    )(periods_embedding, weekend_embedding)
    return fused3.reshape(PERIODS * WEEKEND, OUT_DIM)


def _sc_body(x_hbm, tab_hbm, out_hbm, x_v, idx_v, rows_v, gsem, osem):
    wid = lax.axis_index("s") * 2 + lax.axis_index("c")
    iota = lax.iota(jnp.int32, 16)
    col1 = jnp.full((16,), 1, jnp.int32)
    col2 = jnp.full((16,), 2, jnp.int32)

    def block(j, carry):
        base = wid * PER_W + j * B_BLK
        pltpu.sync_copy(x_hbm.at[pl.ds(base, B_BLK)], x_v)
        for g in range(GROUPS):
            rows = g * 16 + iota
            pv = plsc.load_gather(x_v, [rows, col1])
            wv = plsc.load_gather(x_v, [rows, col2])
            pi = jnp.minimum((pv * float(PERIODS)).astype(jnp.int32), PERIODS - 1)
            wi = jnp.minimum(wv.astype(jnp.int32), WEEKEND - 1)
            idx_v[g // 6, pl.ds((g % 6) * 16, 16)] = pi * WEEKEND + wi
        descs = [
            pltpu.async_copy(
                tab_hbm.at[idx_v.at[r]], rows_v.at[pl.ds(r * SUB, SUB)], gsem
            )
            for r in range(NSUB)
        ]
        for d in descs:
            d.wait()
        pltpu.async_copy(rows_v, out_hbm.at[pl.ds(base, B_BLK)], osem).wait()
        return carry

    lax.fori_loop(0, N_BLK, block, 0)


@jax.jit
def _encode(x_flat, fused_table):
    mesh = plsc.VectorSubcoreMesh(core_axis_name="c", subcore_axis_name="s")
    return pl.kernel(
        _sc_body,
        out_type=jax.ShapeDtypeStruct((N_ROWS, OUT_DIM), jnp.float32),
        mesh=mesh,
        compiler_params=pltpu.CompilerParams(
            needs_layout_passes=False, use_tc_tiling_on_sc=False
        ),
        scratch_types=[
            pltpu.VMEM((B_BLK, 3), jnp.float32),
            pltpu.VMEM((NSUB, SUB), jnp.int32),
            pltpu.VMEM((B_BLK, OUT_DIM), jnp.float32),
            pltpu.SemaphoreType.DMA,
            pltpu.SemaphoreType.DMA,
        ],
    )(x_flat, fused_table)


def kernel(x, periods_embedding, weekend_embedding):
    b, t, n, _ = x.shape
    fused = _build_fused_table(periods_embedding, weekend_embedding)
    x_flat = x.reshape(b * t * n, 3)
    out = _encode(x_flat, fused)
    return out.reshape(b, t, n, OUT_DIM)


# speedup vs baseline: 4.8738x; 3.1926x over previous
import jax
import jax.numpy as jnp
from jax import lax
from jax.experimental import pallas as pl
from jax.experimental.pallas import tpu as pltpu
from jax.experimental.pallas import tpu_sc as plsc

PERIODS = 288
WEEKEND = 7
P_DIM = 24
W_DIM = 24
OUT_DIM = P_DIM + W_DIM
PAD_DIM = 128
N_TAB = PERIODS * WEEKEND
N_ROWS = 64 * 12 * 1000
NW = 32
PER_W = N_ROWS // NW
SUB = 80
NSUB = 2
B_BLK = SUB * NSUB
N_BLK = PER_W // B_BLK
GROUPS = SUB // 16
DEPTH = 3


def _build_table_kernel(p_ref, w_ref, o_ref):
    rows = lax.broadcasted_iota(jnp.int32, (N_TAB, PERIODS), 0)
    cols = lax.broadcasted_iota(jnp.int32, (N_TAB, PERIODS), 1)
    oh_p = (rows // WEEKEND == cols).astype(jnp.float32)
    pemb = jnp.dot(oh_p, p_ref[:], preferred_element_type=jnp.float32)
    rows7 = lax.broadcasted_iota(jnp.int32, (N_TAB, WEEKEND), 0)
    cols7 = lax.broadcasted_iota(jnp.int32, (N_TAB, WEEKEND), 1)
    oh_w = (rows7 % WEEKEND == cols7).astype(jnp.float32)
    wemb = jnp.dot(oh_w, w_ref[:], preferred_element_type=jnp.float32)
    pad = jnp.zeros((N_TAB, PAD_DIM - OUT_DIM), jnp.float32)
    o_ref[:] = jnp.concatenate([pemb, wemb, pad], axis=-1)


def _build_fused_table(periods_embedding, weekend_embedding):
    return pl.pallas_call(
        _build_table_kernel,
        out_shape=jax.ShapeDtypeStruct((N_TAB, PAD_DIM), jnp.float32),
    )(periods_embedding, weekend_embedding)


def _sc_body(
    xp_hbm, xw_hbm, tab_hbm, out_hbm,
    xp0, xp1, xp2, xw0, xw1, xw2, idx0, idx1, idx2, rows0, rows1, rows2,
    xs0, xs1, xs2, gs0, gs1, gs2, os0, os1, os2,
):
    wid = lax.axis_index("s") * 2 + lax.axis_index("c")
    w_base = wid * PER_W
    xp_v = (xp0, xp1, xp2)
    xw_v = (xw0, xw1, xw2)
    idx_v = (idx0, idx1, idx2)
    rows_v = (rows0, rows1, rows2)
    xs = (xs0, xs1, xs2)
    gs = (gs0, gs1, gs2)
    osem = (os0, os1, os2)

    def fire_x(i, s):
        base = w_base + i * B_BLK
        pltpu.async_copy(xp_hbm.at[pl.ds(base, B_BLK)], xp_v[s], xs[s])
        pltpu.async_copy(xw_hbm.at[pl.ds(base, B_BLK)], xw_v[s], xs[s])

    def wait_x(i, s):
        base = w_base + i * B_BLK
        pltpu.make_async_copy(xp_hbm.at[pl.ds(base, B_BLK)], xp_v[s], xs[s]).wait()
        pltpu.make_async_copy(xw_hbm.at[pl.ds(base, B_BLK)], xw_v[s], xs[s]).wait()

    def compute_idx(s):
        for r in range(NSUB):
            for g in range(GROUPS):
                o = r * SUB + g * 16
                pv = xp_v[s][pl.ds(o, 16)]
                wv = xw_v[s][pl.ds(o, 16)]
                pi = jnp.minimum((pv * float(PERIODS)).astype(jnp.int32), PERIODS - 1)
                wi = jnp.minimum(wv.astype(jnp.int32), WEEKEND - 1)
                idx_v[s][r, pl.ds(g * 16, 16)] = pi * WEEKEND + wi

    def fire_gathers(s):
        for r in range(NSUB):
            pltpu.async_copy(
                tab_hbm.at[idx_v[s].at[r]], rows_v[s].at[pl.ds(r * SUB, SUB)], gs[s]
            )

    def wait_gathers(s):
        for r in range(NSUB):
            pltpu.make_async_copy(
                tab_hbm.at[idx_v[s].at[r]], rows_v[s].at[pl.ds(r * SUB, SUB)], gs[s]
            ).wait()

    def fire_out(i, s):
        base = w_base + i * B_BLK
        pltpu.async_copy(rows_v[s], out_hbm.at[pl.ds(base, B_BLK)], osem[s])

    def wait_out(i, s):
        base = w_base + i * B_BLK
        pltpu.make_async_copy(
            rows_v[s], out_hbm.at[pl.ds(base, B_BLK)], osem[s]
        ).wait()

    for s in range(DEPTH):
        fire_x(s, s)
    for i in range(DEPTH):
        s = i % DEPTH
        wait_x(i, s)
        compute_idx(s)
        fire_x(i + DEPTH, s)
        fire_gathers(s)
        if i >= 1:
            wait_gathers(s - 1)
            fire_out(i - 1, s - 1)

    def steady(j, carry):
        i0 = DEPTH * j
        for d in range(DEPTH):
            i = i0 + d
            s = d
            wait_x(i, s)
            compute_idx(s)

            @pl.when(i + DEPTH < N_BLK)
            def _():
                fire_x(i + DEPTH, s)

            wait_out(i - DEPTH, s)
            fire_gathers(s)
            wait_gathers((s - 1) % DEPTH)
            fire_out(i - 1, (s - 1) % DEPTH)
        return carry

    lax.fori_loop(1, N_BLK // DEPTH, steady, 0)

    wait_gathers(DEPTH - 1)
    fire_out(N_BLK - 1, DEPTH - 1)
    for s in range(DEPTH):
        wait_out(N_BLK - DEPTH + s, s)


@jax.jit
def _encode(xp, xw, fused_table):
    mesh = plsc.VectorSubcoreMesh(core_axis_name="c", subcore_axis_name="s")
    return pl.kernel(
        _sc_body,
        out_type=jax.ShapeDtypeStruct((N_ROWS, PAD_DIM), jnp.float32),
        mesh=mesh,
        compiler_params=pltpu.CompilerParams(needs_layout_passes=False),
        scratch_types=(
            [pltpu.VMEM((B_BLK,), jnp.float32) for _ in range(2 * DEPTH)]
            + [pltpu.VMEM((NSUB, SUB), jnp.int32) for _ in range(DEPTH)]
            + [pltpu.VMEM((B_BLK, PAD_DIM), jnp.float32) for _ in range(DEPTH)]
            + [pltpu.SemaphoreType.DMA for _ in range(3 * DEPTH)]
        ),
    )(xp, xw, fused_table)


def kernel(x, periods_embedding, weekend_embedding):
    b, t, n, _ = x.shape
    fused = _build_fused_table(periods_embedding, weekend_embedding)
    xp = x[..., 1].reshape(-1)
    xw = x[..., 2].reshape(-1)
    out = _encode(xp, xw, fused)
    return out[:, :OUT_DIM].reshape(b, t, n, OUT_DIM)


# speedup vs baseline: 9.9143x; 2.0342x over previous
import jax
import jax.numpy as jnp
from jax import lax
from jax.experimental import pallas as pl
from jax.experimental.pallas import tpu as pltpu
from jax.experimental.pallas import tpu_sc as plsc

PERIODS = 288
WEEKEND = 7
P_DIM = 24
W_DIM = 24
OUT_DIM = P_DIM + W_DIM
PAD_DIM = 128
N_TAB = PERIODS * WEEKEND
N_ROWS = 64 * 12 * 1000
NW = 32
PER_W = N_ROWS // NW
SUB = 96
NSUB = 5
B_BLK = SUB * NSUB
N_BLK = PER_W // B_BLK
GROUPS = SUB // 16
DEPTH = 5


def _build_table_kernel(p_ref, w_ref, o_ref):
    pe = jnp.broadcast_to(p_ref[:][:, None, :], (PERIODS, WEEKEND, P_DIM))
    we = jnp.broadcast_to(w_ref[:][None, :, :], (PERIODS, WEEKEND, W_DIM))
    o_ref[:] = jnp.concatenate([pe, we], axis=-1)


def _build_fused_table(periods_embedding, weekend_embedding):
    fused3 = pl.pallas_call(
        _build_table_kernel,
        out_shape=jax.ShapeDtypeStruct((PERIODS, WEEKEND, OUT_DIM), jnp.float32),
    )(periods_embedding, weekend_embedding)
    return fused3.reshape(N_TAB, OUT_DIM)


def _sc_body(xp_hbm, xw_hbm, tab_hbm, out_hbm, xp_v, xw_v, idx_v, rows_v, xs, gs, osem):
    wid = lax.axis_index("s") * 2 + lax.axis_index("c")
    w_base = wid * PER_W

    def fire_x(i, s):
        base = w_base + i * B_BLK
        pltpu.async_copy(xp_hbm.at[pl.ds(base, B_BLK)], xp_v[s], xs[s])
        pltpu.async_copy(xw_hbm.at[pl.ds(base, B_BLK)], xw_v[s], xs[s])

    def wait_x(i, s):
        base = w_base + i * B_BLK
        pltpu.make_async_copy(xp_hbm.at[pl.ds(base, B_BLK)], xp_v[s], xs[s]).wait()
        pltpu.make_async_copy(xw_hbm.at[pl.ds(base, B_BLK)], xw_v[s], xs[s]).wait()

    def compute_idx(s):
        for r in range(NSUB):
            for g in range(GROUPS):
                o = r * SUB + g * 16
                pv = xp_v[s][pl.ds(o, 16)]
                wv = xw_v[s][pl.ds(o, 16)]
                pi = jnp.minimum((pv * float(PERIODS)).astype(jnp.int32), PERIODS - 1)
                wi = jnp.minimum(wv.astype(jnp.int32), WEEKEND - 1)
                idx_v[s][r, pl.ds(g * 16, 16)] = pi * WEEKEND + wi

    def fire_gathers(s):
        for r in range(NSUB):
            pltpu.async_copy(
                tab_hbm.at[idx_v[s].at[r]], rows_v[s].at[pl.ds(r * SUB, SUB)], gs[s]
            )

    def wait_gathers(s):
        for r in range(NSUB):
            pltpu.make_async_copy(
                tab_hbm.at[idx_v[s].at[r]], rows_v[s].at[pl.ds(r * SUB, SUB)], gs[s]
            ).wait()

    def fire_out(i, s):
        base = w_base + i * B_BLK
        pltpu.async_copy(
            rows_v[s], out_hbm.at[pl.ds(base, B_BLK), pl.ds(0, OUT_DIM)], osem[s]
        )

    def wait_out(i, s):
        base = w_base + i * B_BLK
        pltpu.make_async_copy(
            rows_v[s], out_hbm.at[pl.ds(base, B_BLK), pl.ds(0, OUT_DIM)], osem[s]
        ).wait()

    for s in range(DEPTH):
        fire_x(s, s)
    for i in range(DEPTH):
        wait_x(i, i)
        compute_idx(i)
        fire_x(i + DEPTH, i)
        fire_gathers(i)
        if i >= 2:
            wait_gathers(i - 2)
            fire_out(i - 2, i - 2)

    def steady(j, carry):
        i0 = DEPTH * j
        for d in range(DEPTH):
            i = i0 + d
            s = d
            wait_x(i, s)
            compute_idx(s)

            @pl.when(i + DEPTH < N_BLK)
            def _():
                fire_x(i + DEPTH, s)

            wait_out(i - DEPTH, s)
            fire_gathers(s)
            wait_gathers((s - 2) % DEPTH)
            fire_out(i - 2, (s - 2) % DEPTH)
        return carry

    lax.fori_loop(1, N_BLK // DEPTH, steady, 0)

    for i in (N_BLK - 2, N_BLK - 1):
        wait_gathers(i % DEPTH)
        fire_out(i, i % DEPTH)
    for s in range(DEPTH):
        wait_out(N_BLK - DEPTH + s, s)


@jax.jit
def _encode(xp, xw, fused_table):
    mesh = plsc.VectorSubcoreMesh(core_axis_name="c", subcore_axis_name="s")
    return pl.kernel(
        _sc_body,
        out_type=jax.ShapeDtypeStruct((N_ROWS, PAD_DIM), jnp.float32),
        mesh=mesh,
        compiler_params=pltpu.CompilerParams(
            needs_layout_passes=False, use_tc_tiling_on_sc=False
        ),
        scratch_types=dict(
            xp_v=[pltpu.VMEM((B_BLK,), jnp.float32) for _ in range(DEPTH)],
            xw_v=[pltpu.VMEM((B_BLK,), jnp.float32) for _ in range(DEPTH)],
            idx_v=[pltpu.VMEM((NSUB, SUB), jnp.int32) for _ in range(DEPTH)],
            rows_v=[pltpu.VMEM((B_BLK, OUT_DIM), jnp.float32) for _ in range(DEPTH)],
            xs=[pltpu.SemaphoreType.DMA for _ in range(DEPTH)],
            gs=[pltpu.SemaphoreType.DMA for _ in range(DEPTH)],
            osem=[pltpu.SemaphoreType.DMA for _ in range(DEPTH)],
        ),
    )(xp, xw, fused_table)


def kernel(x, periods_embedding, weekend_embedding):
    b, t, n, _ = x.shape
    fused = _build_fused_table(periods_embedding, weekend_embedding)
    xp = x[..., 1].reshape(-1)
    xw = x[..., 2].reshape(-1)
    out = _encode(xp, xw, fused)
    return out[:, :OUT_DIM].reshape(b, t, n, OUT_DIM)
